# Initial kernel scaffold; baseline (speedup 1.0000x reference)
import jax
import jax.numpy as jnp
from jax.experimental import pallas as pl


def kernel(z):
    raise NotImplementedError("write your pallas kernel here")


# speedup vs baseline: 4.1933x; 4.1933x over previous
import jax
import jax.numpy as jnp
from jax import lax
from jax.experimental import pallas as pl
from jax.experimental.pallas import tpu as pltpu
from jax.experimental.pallas import tpu_sc as plsc

K = 64
ROWS = 128
N = 32768
NVEC = N // 16
NW = 32
ROWS_PER = ROWS // NW
SAMPLE_TARGET = 16
I32MIN = -2147483648


def _kernel_body(z_hbm, out_hbm, zbuf, candidx, hist, selidx, outbuf, sem_out):
    iota = lax.iota(jnp.int32, 16)
    ones = jnp.ones((16,), jnp.int32)
    zeros_i = jnp.zeros((16,), jnp.int32)
    zeros_f = jnp.zeros((16,), jnp.float32)
    stripe8 = iota * 256
    stripe4 = iota * 16

    def key_of(zv):
        i = lax.bitcast_convert_type(zv, jnp.int32)
        return jnp.where(i >= 0, i, I32MIN - i)

    def float_of(tk):
        i = jnp.where(tk >= 0, tk, I32MIN - tk)
        return lax.bitcast_convert_type(i, jnp.float32)

    def clear_hist():
        def body(m, _):
            hist[pl.ds(m * 16, 16)] = zeros_i
            return 0
        lax.fori_loop(0, 256, body, 0)

    def analyze8(k):
        def body(j, carry):
            bmax, cab, acc_above = carry
            i = 15 - j
            h = zeros_i
            for l in range(16):
                h = h + hist[pl.ds(l * 256 + i * 16, 16)]
                hist[pl.ds(l * 256 + i * 16, 16)] = zeros_i
            suffix = lax.rev(plsc.cumsum(lax.rev(h, (0,))), (0,))
            tot_ge = suffix + acc_above
            bins = i * 16 + iota
            bmax = jnp.maximum(bmax, jnp.max(jnp.where(tot_ge >= k, bins, -1)))
            cab = jnp.maximum(cab, jnp.max(jnp.where(tot_ge < k, tot_ge, 0)))
            return bmax, cab, acc_above + jnp.sum(h)
        bmax, cab, _ = lax.fori_loop(
            0, 16, body, (jnp.int32(-1), jnp.int32(0), jnp.int32(0)))
        return bmax, k - cab

    def analyze4(k):
        h = zeros_i
        for l in range(16):
            h = h + hist[pl.ds(l * 16, 16)]
            hist[pl.ds(l * 16, 16)] = zeros_i
        suffix = lax.rev(plsc.cumsum(lax.rev(h, (0,))), (0,))
        bmax = jnp.max(jnp.where(suffix >= k, iota, -1))
        cab = jnp.max(jnp.where(suffix < k, suffix, 0))
        return bmax, k - cab

    def compact(b):
        flo = float_of(jnp.full((16,), (b - 128) << 24, jnp.int32))
        def body(j, cnt):
            zv = zbuf[pl.ds(j * 16, 16)]
            m = zv >= flo
            mi = m.astype(jnp.int32)
            pos = cnt + plsc.cumsum(mi) - 1
            plsc.store_scatter(candidx, [pos], j * 16 + iota, mask=m)
            return cnt + jnp.sum(mi)
        return lax.fori_loop(0, NVEC, body, jnp.int32(0))

    wid = lax.axis_index("s") * 2 + lax.axis_index("c")

    def zinit(m, _):
        outbuf[pl.ds(m * 16, 16)] = zeros_f
        return 0
    lax.fori_loop(0, NVEC, zinit, 0)
    clear_hist()

    row0 = wid * ROWS_PER
    out_desc = None

    for r in range(ROWS_PER):
        row = row0 + r
        pltpu.sync_copy(z_hbm.at[pl.ds(row * N, N)], zbuf)

        def sample(j, _):
            key = key_of(zbuf[pl.ds(j * 256, 16)])
            b = (key >> 24) + 128
            plsc.addupdate_scatter(hist, [stripe8 + b], ones)
            return 0
        lax.fori_loop(0, NVEC // 16, sample, 0)
        b_s, _ = analyze8(jnp.int32(SAMPLE_TARGET))

        n_cand = compact(b_s)
        def fallback(carry):
            b, _ = carry
            return b - 1, compact(b - 1)
        b_s, n_cand = lax.while_loop(
            lambda c: c[1] < K, fallback, (b_s, n_cand))
        nv = (n_cand + 15) // 16

        def cp8(j, _):
            valid = (j * 16 + iota) < n_cand
            idx = candidx[pl.ds(j * 16, 16)]
            key = key_of(plsc.load_gather(zbuf, [idx], mask=valid))
            b = (key >> 24) + 128
            plsc.addupdate_scatter(hist, [stripe8 + b], ones, mask=valid)
            return 0
        lax.fori_loop(0, nv, cp8, 0)
        b1, k = analyze8(jnp.int32(K))
        prefix = b1 - 128

        for sh in (20, 16, 12, 8, 4, 0):
            def cp4(j, _, sh=sh, prefix=prefix):
                valid = (j * 16 + iota) < n_cand
                idx = candidx[pl.ds(j * 16, 16)]
                key = key_of(plsc.load_gather(zbuf, [idx], mask=valid))
                match = valid & ((key >> (sh + 4)) == prefix)
                b = (key >> sh) & 15
                plsc.addupdate_scatter(hist, [stripe4 + b], ones, mask=match)
                return 0
            lax.fori_loop(0, nv, cp4, 0)
            b4, k = analyze4(k)
            prefix = (prefix << 4) | b4
        tkey, need = prefix, k

        if out_desc is not None:
            out_desc.wait()
            for m in range(K // 16):
                si = selidx[pl.ds(m * 16, 16)]
                plsc.store_scatter(outbuf, [si], zeros_f)

        ft = float_of(jnp.full((16,), tkey, jnp.int32))
        def ap(j, carry):
            run, scnt = carry
            valid = (j * 16 + iota) < n_cand
            idx = candidx[pl.ds(j * 16, 16)]
            zv = plsc.load_gather(zbuf, [idx], mask=valid)
            gt = valid & (zv > ft)
            eq = valid & (zv == ft)
            eqi = eq.astype(jnp.int32)
            sel = gt | (eq & ((run + plsc.cumsum(eqi)) <= need))
            seli = sel.astype(jnp.int32)
            plsc.store_scatter(outbuf, [idx], zv, mask=sel)
            spos = scnt + plsc.cumsum(seli) - 1
            plsc.store_scatter(selidx, [spos], idx, mask=sel)
            return run + jnp.sum(eqi), scnt + jnp.sum(seli)
        lax.fori_loop(0, nv, ap, (jnp.int32(0), jnp.int32(0)))

        out_desc = pltpu.async_copy(outbuf, out_hbm.at[pl.ds(row * N, N)],
                                    sem_out)

    out_desc.wait()


@jax.jit
def kernel(z):
    mesh = plsc.VectorSubcoreMesh(core_axis_name="c", subcore_axis_name="s")
    call = pl.kernel(
        _kernel_body,
        out_type=jax.ShapeDtypeStruct((ROWS * N,), jnp.float32),
        mesh=mesh,
        compiler_params=pltpu.CompilerParams(needs_layout_passes=False),
        scratch_types=[
            pltpu.VMEM((N,), jnp.float32),
            pltpu.VMEM((N,), jnp.int32),
            pltpu.VMEM((4096,), jnp.int32),
            pltpu.VMEM((K,), jnp.int32),
            pltpu.VMEM((N,), jnp.float32),
            pltpu.SemaphoreType.DMA,
        ],
    )
    return call(z.reshape(ROWS * N)).reshape(ROWS, N)
